# baseline (speedup 1.0000x reference)
import functools

import jax
import jax.numpy as jnp
from jax import lax
from jax.experimental import pallas as pl
from jax.experimental.pallas import tpu as pltpu
from jax.experimental.pallas import tpu_sc as plsc

NUM_EXPERTS = 8
HIDDEN = 1024
INTERMEDIATE = 4096
TOKENS = 2048
JITTER_EPS = 0.01

BLK = 256
PSLOTS = 4096 + NUM_EXPERTS * BLK
NBLK = PSLOTS // BLK
BE_PAD = 64

N_CHUNK = 256
NI = INTERMEDIATE // N_CHUNK

NW = 32
TPW = TOKENS // NW


def _cumsum_tokens(arr):
    n = arr.shape[0]
    d = 1
    while d < n:
        shifted = jnp.concatenate(
            [jnp.zeros((d, arr.shape[1]), arr.dtype), arr[:-d]], axis=0)
        arr = arr + shifted
        d *= 2
    return arr


def _routing_kernel(scores_ref, pos0_ref, pos1_ref, m0_ref, m1_ref, be_ref):
    scores = scores_ref[...]
    ids = lax.broadcasted_iota(jnp.int32, scores.shape, 1)
    neg_inf = jnp.float32(-jnp.inf)

    max1 = jnp.max(scores, axis=-1, keepdims=True)
    ind1 = jnp.min(jnp.where(scores == max1, ids, NUM_EXPERTS), axis=-1,
                   keepdims=True)
    factor = jnp.maximum(jnp.abs(scores), max1)
    mask1 = (max1 - scores) / factor > 2.0 * JITTER_EPS
    p1 = jnp.exp(jnp.where(mask1, neg_inf, scores) - max1)
    oh1 = ids == ind1
    mult1 = (jnp.sum(jnp.where(oh1, p1, 0.0), axis=-1, keepdims=True)
             / jnp.sum(p1, axis=-1, keepdims=True))

    masked_scores = jnp.where(oh1, neg_inf, scores)
    max2 = jnp.max(masked_scores, axis=-1, keepdims=True)
    ind2 = jnp.min(jnp.where(masked_scores == max2, ids, NUM_EXPERTS),
                   axis=-1, keepdims=True)
    factor2 = jnp.maximum(jnp.abs(scores), max2)
    mask2 = (max2 - scores) / factor2 > 2.0 * JITTER_EPS
    p2 = jnp.exp(jnp.where(mask2, neg_inf, masked_scores) - max2)
    oh2 = ids == ind2
    mult2 = (jnp.sum(jnp.where(oh2, p2, 0.0), axis=-1, keepdims=True)
             / jnp.sum(p2, axis=-1, keepdims=True))

    f1 = oh1.astype(jnp.float32)
    f2 = oh2.astype(jnp.float32)
    c1 = _cumsum_tokens(f1)
    c2 = _cumsum_tokens(f2)
    cnt1 = c1[TOKENS - 1:TOKENS, :]
    cnt2 = c2[TOKENS - 1:TOKENS, :]
    rank0 = c1 - f1
    rank1 = (c2 - f2) + cnt1
    counts = cnt1 + cnt2
    pc = jnp.ceil(counts / BLK) * BLK
    inc = pc
    d = 1
    while d < NUM_EXPERTS:
        inc = inc + jnp.concatenate(
            [jnp.zeros((1, d), jnp.float32), inc[:, :-d]], axis=1)
        d *= 2
    off = inc - pc
    total = jnp.sum(pc)

    pos0 = jnp.sum(f1 * (off + rank0), axis=-1, keepdims=True)
    pos1 = jnp.sum(f2 * (off + rank1), axis=-1, keepdims=True)
    pos0_ref[...] = pos0.astype(jnp.int32)
    pos1_ref[...] = pos1.astype(jnp.int32)
    m0_ref[...] = mult1
    m1_ref[...] = mult2

    bid = lax.broadcasted_iota(jnp.int32, (BE_PAD, NUM_EXPERTS), 0)
    rowstart = (bid * BLK).astype(jnp.float32)
    cntge = jnp.sum((rowstart >= off).astype(jnp.int32), axis=-1,
                    keepdims=True)
    active = rowstart[:, :1] < total
    be_ref[...] = jnp.where(active, cntge - 1, -1)


def _routing(scores):
    return pl.pallas_call(
        _routing_kernel,
        out_shape=(
            jax.ShapeDtypeStruct((TOKENS, 1), jnp.int32),
            jax.ShapeDtypeStruct((TOKENS, 1), jnp.int32),
            jax.ShapeDtypeStruct((TOKENS, 1), jnp.float32),
            jax.ShapeDtypeStruct((TOKENS, 1), jnp.float32),
            jax.ShapeDtypeStruct((BE_PAD, 1), jnp.int32),
        ),
    )(scores)


def _sc_dispatch(x, pos0, pos1):
    mesh = plsc.VectorSubcoreMesh(core_axis_name="c", subcore_axis_name="s")

    @functools.partial(
        pl.kernel, mesh=mesh,
        out_type=jax.ShapeDtypeStruct((PSLOTS, HIDDEN), jnp.float32),
        scratch_types=[
            pltpu.VMEM((TPW,), jnp.int32),
            pltpu.VMEM((TPW,), jnp.int32),
            pltpu.VMEM((TPW, HIDDEN), jnp.float32),
            pltpu.SemaphoreType.DMA,
        ],
    )
    def k(x_hbm, p0_hbm, p1_hbm, xs_hbm, i0, i1, xb, sem):
        wid = lax.axis_index("s") * 2 + lax.axis_index("c")
        base = wid * TPW
        pltpu.sync_copy(x_hbm.at[pl.ds(base, TPW)], xb)
        pltpu.sync_copy(p0_hbm.at[pl.ds(base, TPW)], i0)
        pltpu.sync_copy(p1_hbm.at[pl.ds(base, TPW)], i1)
        pltpu.async_copy(xb, xs_hbm.at[i0], sem).wait()
        pltpu.async_copy(xb, xs_hbm.at[i1], sem).wait()

    return k(x, pos0, pos1)


def _grouped_kernel(be_ref, xs_ref, w1_ref, w3_ref, w2_ref, ys_ref):
    n = pl.program_id(0)
    b = pl.program_id(1)
    e = be_ref[b]

    @pl.when(e >= 0)
    def _():
        row = pl.multiple_of(b * BLK, BLK)
        xb = xs_ref[pl.ds(row, BLK), :]
        gate = lax.dot_general(xb, w1_ref[0].astype(jnp.bfloat16),
                               (((1,), (1,)), ((), ())),
                               preferred_element_type=jnp.float32)
        up = lax.dot_general(xb, w3_ref[0].astype(jnp.bfloat16),
                             (((1,), (1,)), ((), ())),
                             preferred_element_type=jnp.float32)
        h = (gate * jax.nn.sigmoid(gate) * up).astype(jnp.bfloat16)
        part = lax.dot_general(h, w2_ref[0].astype(jnp.bfloat16),
                               (((1,), (1,)), ((), ())),
                               preferred_element_type=jnp.float32)

        @pl.when(n == 0)
        def _w():
            ys_ref[pl.ds(row, BLK), :] = part

        @pl.when(n > 0)
        def _a():
            ys_ref[pl.ds(row, BLK), :] += part


def _grouped(be, xs, w1, w3, w2):
    grid_spec = pltpu.PrefetchScalarGridSpec(
        num_scalar_prefetch=1,
        grid=(NI, NBLK),
        in_specs=[
            pl.BlockSpec((PSLOTS, HIDDEN), lambda n, b, be: (0, 0)),
            pl.BlockSpec((1, N_CHUNK, HIDDEN),
                         lambda n, b, be: (jnp.maximum(be[b], 0), n, 0)),
            pl.BlockSpec((1, N_CHUNK, HIDDEN),
                         lambda n, b, be: (jnp.maximum(be[b], 0), n, 0)),
            pl.BlockSpec((1, HIDDEN, N_CHUNK),
                         lambda n, b, be: (jnp.maximum(be[b], 0), 0, n)),
        ],
        out_specs=pl.BlockSpec((PSLOTS, HIDDEN), lambda n, b, be: (0, 0)),
    )
    return pl.pallas_call(
        _grouped_kernel,
        grid_spec=grid_spec,
        out_shape=jax.ShapeDtypeStruct((PSLOTS, HIDDEN), jnp.float32),
        compiler_params=pltpu.CompilerParams(
            dimension_semantics=("arbitrary", "arbitrary"),
        ),
    )(be, xs, w1, w3, w2)


def _sc_combine(ys, pos0, pos1):
    mesh = plsc.VectorSubcoreMesh(core_axis_name="c", subcore_axis_name="s")

    @functools.partial(
        pl.kernel, mesh=mesh,
        out_type=(
            jax.ShapeDtypeStruct((TOKENS, HIDDEN), jnp.float32),
            jax.ShapeDtypeStruct((TOKENS, HIDDEN), jnp.float32),
        ),
        scratch_types=[
            pltpu.VMEM((TPW,), jnp.int32),
            pltpu.VMEM((TPW,), jnp.int32),
            pltpu.VMEM((TPW, HIDDEN), jnp.float32),
            pltpu.SemaphoreType.DMA,
        ],
    )
    def k(ys_hbm, p0_hbm, p1_hbm, g0_hbm, g1_hbm, i0, i1, buf, sem):
        wid = lax.axis_index("s") * 2 + lax.axis_index("c")
        base = wid * TPW
        pltpu.sync_copy(p0_hbm.at[pl.ds(base, TPW)], i0)
        pltpu.async_copy(ys_hbm.at[i0], buf, sem).wait()
        pltpu.sync_copy(buf, g0_hbm.at[pl.ds(base, TPW)])
        pltpu.sync_copy(p1_hbm.at[pl.ds(base, TPW)], i1)
        pltpu.async_copy(ys_hbm.at[i1], buf, sem).wait()
        pltpu.sync_copy(buf, g1_hbm.at[pl.ds(base, TPW)])

    return k(ys, pos0, pos1)


def _combine_kernel(g0_ref, g1_ref, m0_ref, m1_ref, out_ref):
    out_ref[...] = m0_ref[...] * g0_ref[...] + m1_ref[...] * g1_ref[...]


def _combine(g0, g1, m0, m1):
    tb = 256
    return pl.pallas_call(
        _combine_kernel,
        grid=(TOKENS // tb,),
        in_specs=[
            pl.BlockSpec((tb, HIDDEN), lambda i: (i, 0)),
            pl.BlockSpec((tb, HIDDEN), lambda i: (i, 0)),
            pl.BlockSpec((tb, 1), lambda i: (i, 0)),
            pl.BlockSpec((tb, 1), lambda i: (i, 0)),
        ],
        out_specs=pl.BlockSpec((tb, HIDDEN), lambda i: (i, 0)),
        out_shape=jax.ShapeDtypeStruct((TOKENS, HIDDEN), jnp.float32),
    )(g0, g1, m0, m1)


def kernel(hidden_states, w_gate, w1, w2, w3):
    x = hidden_states.reshape(-1, HIDDEN)
    scores = x @ w_gate.T
    pos0c, pos1c, m0, m1, bec = _routing(scores)
    pos0 = pos0c.reshape(TOKENS)
    pos1 = pos1c.reshape(TOKENS)
    be = bec.reshape(BE_PAD)[:NBLK]
    xs = _sc_dispatch(x, pos0, pos1)
    ys = _grouped(be, xs.astype(jnp.bfloat16), w1, w3, w2)
    g0, g1 = _sc_combine(ys, pos0, pos1)
    out = _combine(g0, g1, m0, m1)
    return out.reshape(hidden_states.shape)


# speedup vs baseline: 1.4640x; 1.4640x over previous
import functools

import jax
import jax.numpy as jnp
from jax import lax
from jax.experimental import pallas as pl
from jax.experimental.pallas import tpu as pltpu
from jax.experimental.pallas import tpu_sc as plsc

NUM_EXPERTS = 8
HIDDEN = 1024
INTERMEDIATE = 4096
TOKENS = 2048
JITTER_EPS = 0.01

BLK = 256
PSLOTS = 4096 + NUM_EXPERTS * BLK
NBLK = PSLOTS // BLK
BE_PAD = 64

N_CHUNK = 1024
NI = INTERMEDIATE // N_CHUNK

NW = 32
TPW = TOKENS // NW


def _cumsum_tokens(arr):
    n = arr.shape[0]
    d = 1
    while d < n:
        shifted = jnp.concatenate(
            [jnp.zeros((d, arr.shape[1]), arr.dtype), arr[:-d]], axis=0)
        arr = arr + shifted
        d *= 2
    return arr


def _routing_kernel(scores_ref, pos0_ref, pos1_ref, m0_ref, m1_ref, be_ref):
    scores = scores_ref[...]
    ids = lax.broadcasted_iota(jnp.int32, scores.shape, 1)
    neg_inf = jnp.float32(-jnp.inf)

    max1 = jnp.max(scores, axis=-1, keepdims=True)
    ind1 = jnp.min(jnp.where(scores == max1, ids, NUM_EXPERTS), axis=-1,
                   keepdims=True)
    factor = jnp.maximum(jnp.abs(scores), max1)
    mask1 = (max1 - scores) / factor > 2.0 * JITTER_EPS
    p1 = jnp.exp(jnp.where(mask1, neg_inf, scores) - max1)
    oh1 = ids == ind1
    mult1 = (jnp.sum(jnp.where(oh1, p1, 0.0), axis=-1, keepdims=True)
             / jnp.sum(p1, axis=-1, keepdims=True))

    masked_scores = jnp.where(oh1, neg_inf, scores)
    max2 = jnp.max(masked_scores, axis=-1, keepdims=True)
    ind2 = jnp.min(jnp.where(masked_scores == max2, ids, NUM_EXPERTS),
                   axis=-1, keepdims=True)
    factor2 = jnp.maximum(jnp.abs(scores), max2)
    mask2 = (max2 - scores) / factor2 > 2.0 * JITTER_EPS
    p2 = jnp.exp(jnp.where(mask2, neg_inf, masked_scores) - max2)
    oh2 = ids == ind2
    mult2 = (jnp.sum(jnp.where(oh2, p2, 0.0), axis=-1, keepdims=True)
             / jnp.sum(p2, axis=-1, keepdims=True))

    f1 = oh1.astype(jnp.float32)
    f2 = oh2.astype(jnp.float32)
    c1 = _cumsum_tokens(f1)
    c2 = _cumsum_tokens(f2)
    cnt1 = c1[TOKENS - 1:TOKENS, :]
    cnt2 = c2[TOKENS - 1:TOKENS, :]
    rank0 = c1 - f1
    rank1 = (c2 - f2) + cnt1
    counts = cnt1 + cnt2
    pc = jnp.ceil(counts / BLK) * BLK
    inc = pc
    d = 1
    while d < NUM_EXPERTS:
        inc = inc + jnp.concatenate(
            [jnp.zeros((1, d), jnp.float32), inc[:, :-d]], axis=1)
        d *= 2
    off = inc - pc
    total = jnp.sum(pc)

    pos0 = jnp.sum(f1 * (off + rank0), axis=-1, keepdims=True)
    pos1 = jnp.sum(f2 * (off + rank1), axis=-1, keepdims=True)
    pos0_ref[...] = pos0.astype(jnp.int32)
    pos1_ref[...] = pos1.astype(jnp.int32)
    m0_ref[...] = mult1
    m1_ref[...] = mult2

    bid = lax.broadcasted_iota(jnp.int32, (BE_PAD, NUM_EXPERTS), 0)
    rowstart = (bid * BLK).astype(jnp.float32)
    cntge = jnp.sum((rowstart >= off).astype(jnp.int32), axis=-1,
                    keepdims=True)
    active = rowstart[:, :1] < total
    be_ref[...] = jnp.where(active, cntge - 1, -1)


def _routing(scores):
    return pl.pallas_call(
        _routing_kernel,
        out_shape=(
            jax.ShapeDtypeStruct((TOKENS, 1), jnp.int32),
            jax.ShapeDtypeStruct((TOKENS, 1), jnp.int32),
            jax.ShapeDtypeStruct((TOKENS, 1), jnp.float32),
            jax.ShapeDtypeStruct((TOKENS, 1), jnp.float32),
            jax.ShapeDtypeStruct((BE_PAD, 1), jnp.int32),
        ),
    )(scores)


def _sc_dispatch(x, pos0, pos1):
    mesh = plsc.VectorSubcoreMesh(core_axis_name="c", subcore_axis_name="s")

    @functools.partial(
        pl.kernel, mesh=mesh,
        out_type=jax.ShapeDtypeStruct((PSLOTS, HIDDEN), jnp.float32),
        scratch_types=[
            pltpu.VMEM((TPW,), jnp.int32),
            pltpu.VMEM((TPW,), jnp.int32),
            pltpu.VMEM((TPW, HIDDEN), jnp.float32),
            pltpu.SemaphoreType.DMA,
        ],
    )
    def k(x_hbm, p0_hbm, p1_hbm, xs_hbm, i0, i1, xb, sem):
        wid = lax.axis_index("s") * 2 + lax.axis_index("c")
        base = wid * TPW
        pltpu.sync_copy(x_hbm.at[pl.ds(base, TPW)], xb)
        pltpu.sync_copy(p0_hbm.at[pl.ds(base, TPW)], i0)
        pltpu.sync_copy(p1_hbm.at[pl.ds(base, TPW)], i1)
        pltpu.async_copy(xb, xs_hbm.at[i0], sem).wait()
        pltpu.async_copy(xb, xs_hbm.at[i1], sem).wait()

    return k(x, pos0, pos1)


def _grouped_kernel(be_ref, xs_ref, w1_ref, w3_ref, w2_ref, ys_ref):
    n = pl.program_id(0)
    b = pl.program_id(1)
    e = be_ref[b]

    @pl.when(e >= 0)
    def _():
        row = pl.multiple_of(b * BLK, BLK)
        xb = xs_ref[...]
        gate = lax.dot_general(xb, w1_ref[0].astype(jnp.bfloat16),
                               (((1,), (1,)), ((), ())),
                               preferred_element_type=jnp.float32)
        up = lax.dot_general(xb, w3_ref[0].astype(jnp.bfloat16),
                             (((1,), (1,)), ((), ())),
                             preferred_element_type=jnp.float32)
        h = (gate * jax.nn.sigmoid(gate) * up).astype(jnp.bfloat16)
        part = lax.dot_general(h, w2_ref[0].astype(jnp.bfloat16),
                               (((1,), (1,)), ((), ())),
                               preferred_element_type=jnp.float32)

        @pl.when(n == 0)
        def _w():
            ys_ref[pl.ds(row, BLK), :] = part

        @pl.when(n > 0)
        def _a():
            ys_ref[pl.ds(row, BLK), :] += part


def _grouped(be, xs, w1, w3, w2):
    grid_spec = pltpu.PrefetchScalarGridSpec(
        num_scalar_prefetch=1,
        grid=(NI, NBLK),
        in_specs=[
            pl.BlockSpec((BLK, HIDDEN), lambda n, b, be: (b, 0)),
            pl.BlockSpec((1, N_CHUNK, HIDDEN),
                         lambda n, b, be: (jnp.maximum(be[b], 0), n, 0)),
            pl.BlockSpec((1, N_CHUNK, HIDDEN),
                         lambda n, b, be: (jnp.maximum(be[b], 0), n, 0)),
            pl.BlockSpec((1, HIDDEN, N_CHUNK),
                         lambda n, b, be: (jnp.maximum(be[b], 0), 0, n)),
        ],
        out_specs=pl.BlockSpec((PSLOTS, HIDDEN), lambda n, b, be: (0, 0)),
    )
    return pl.pallas_call(
        _grouped_kernel,
        grid_spec=grid_spec,
        out_shape=jax.ShapeDtypeStruct((PSLOTS, HIDDEN), jnp.float32),
        compiler_params=pltpu.CompilerParams(
            dimension_semantics=("arbitrary", "arbitrary"),
        ),
    )(be, xs, w1, w3, w2)


def _sc_combine(ys, pos0, pos1):
    mesh = plsc.VectorSubcoreMesh(core_axis_name="c", subcore_axis_name="s")

    @functools.partial(
        pl.kernel, mesh=mesh,
        out_type=(
            jax.ShapeDtypeStruct((TOKENS, HIDDEN), jnp.float32),
            jax.ShapeDtypeStruct((TOKENS, HIDDEN), jnp.float32),
        ),
        scratch_types=[
            pltpu.VMEM((TPW,), jnp.int32),
            pltpu.VMEM((TPW,), jnp.int32),
            pltpu.VMEM((TPW, HIDDEN), jnp.float32),
            pltpu.SemaphoreType.DMA,
        ],
    )
    def k(ys_hbm, p0_hbm, p1_hbm, g0_hbm, g1_hbm, i0, i1, buf, sem):
        wid = lax.axis_index("s") * 2 + lax.axis_index("c")
        base = wid * TPW
        pltpu.sync_copy(p0_hbm.at[pl.ds(base, TPW)], i0)
        pltpu.async_copy(ys_hbm.at[i0], buf, sem).wait()
        pltpu.sync_copy(buf, g0_hbm.at[pl.ds(base, TPW)])
        pltpu.sync_copy(p1_hbm.at[pl.ds(base, TPW)], i1)
        pltpu.async_copy(ys_hbm.at[i1], buf, sem).wait()
        pltpu.sync_copy(buf, g1_hbm.at[pl.ds(base, TPW)])

    return k(ys, pos0, pos1)


def _combine_kernel(g0_ref, g1_ref, m0_ref, m1_ref, out_ref):
    out_ref[...] = m0_ref[...] * g0_ref[...] + m1_ref[...] * g1_ref[...]


def _combine(g0, g1, m0, m1):
    tb = 256
    return pl.pallas_call(
        _combine_kernel,
        grid=(TOKENS // tb,),
        in_specs=[
            pl.BlockSpec((tb, HIDDEN), lambda i: (i, 0)),
            pl.BlockSpec((tb, HIDDEN), lambda i: (i, 0)),
            pl.BlockSpec((tb, 1), lambda i: (i, 0)),
            pl.BlockSpec((tb, 1), lambda i: (i, 0)),
        ],
        out_specs=pl.BlockSpec((tb, HIDDEN), lambda i: (i, 0)),
        out_shape=jax.ShapeDtypeStruct((TOKENS, HIDDEN), jnp.float32),
    )(g0, g1, m0, m1)


def kernel(hidden_states, w_gate, w1, w2, w3):
    x = hidden_states.reshape(-1, HIDDEN)
    scores = x @ w_gate.T
    pos0c, pos1c, m0, m1, bec = _routing(scores)
    pos0 = pos0c.reshape(TOKENS)
    pos1 = pos1c.reshape(TOKENS)
    be = bec.reshape(BE_PAD)[:NBLK]
    xs = _sc_dispatch(x, pos0, pos1)
    ys = _grouped(be, xs.astype(jnp.bfloat16), w1, w3, w2)
    g0, g1 = _sc_combine(ys, pos0, pos1)
    out = _combine(g0, g1, m0, m1)
    return out.reshape(hidden_states.shape)


# speedup vs baseline: 1.4889x; 1.0170x over previous
import functools

import jax
import jax.numpy as jnp
from jax import lax
from jax.experimental import pallas as pl
from jax.experimental.pallas import tpu as pltpu
from jax.experimental.pallas import tpu_sc as plsc

NUM_EXPERTS = 8
HIDDEN = 1024
INTERMEDIATE = 4096
TOKENS = 2048
JITTER_EPS = 0.01

BLK = 256
PSLOTS = 4096 + NUM_EXPERTS * BLK
NBLK = PSLOTS // BLK
BE_PAD = 64

N_CHUNK = 1024
NI = INTERMEDIATE // N_CHUNK

NW = 32
TPW = TOKENS // NW


def _cumsum_tokens(arr):
    n = arr.shape[0]
    d = 1
    while d < n:
        shifted = jnp.concatenate(
            [jnp.zeros((d, arr.shape[1]), arr.dtype), arr[:-d]], axis=0)
        arr = arr + shifted
        d *= 2
    return arr


def _routing_kernel(scores_ref, pos0_ref, pos1_ref, m0_ref, m1_ref, be_ref):
    scores = scores_ref[...]
    ids = lax.broadcasted_iota(jnp.int32, scores.shape, 1)
    neg_inf = jnp.float32(-jnp.inf)

    max1 = jnp.max(scores, axis=-1, keepdims=True)
    ind1 = jnp.min(jnp.where(scores == max1, ids, NUM_EXPERTS), axis=-1,
                   keepdims=True)
    factor = jnp.maximum(jnp.abs(scores), max1)
    mask1 = (max1 - scores) / factor > 2.0 * JITTER_EPS
    p1 = jnp.exp(jnp.where(mask1, neg_inf, scores) - max1)
    oh1 = ids == ind1
    mult1 = (jnp.sum(jnp.where(oh1, p1, 0.0), axis=-1, keepdims=True)
             / jnp.sum(p1, axis=-1, keepdims=True))

    masked_scores = jnp.where(oh1, neg_inf, scores)
    max2 = jnp.max(masked_scores, axis=-1, keepdims=True)
    ind2 = jnp.min(jnp.where(masked_scores == max2, ids, NUM_EXPERTS),
                   axis=-1, keepdims=True)
    factor2 = jnp.maximum(jnp.abs(scores), max2)
    mask2 = (max2 - scores) / factor2 > 2.0 * JITTER_EPS
    p2 = jnp.exp(jnp.where(mask2, neg_inf, masked_scores) - max2)
    oh2 = ids == ind2
    mult2 = (jnp.sum(jnp.where(oh2, p2, 0.0), axis=-1, keepdims=True)
             / jnp.sum(p2, axis=-1, keepdims=True))

    f1 = oh1.astype(jnp.float32)
    f2 = oh2.astype(jnp.float32)
    c1 = _cumsum_tokens(f1)
    c2 = _cumsum_tokens(f2)
    cnt1 = c1[TOKENS - 1:TOKENS, :]
    cnt2 = c2[TOKENS - 1:TOKENS, :]
    rank0 = c1 - f1
    rank1 = (c2 - f2) + cnt1
    counts = cnt1 + cnt2
    pc = jnp.ceil(counts / BLK) * BLK
    inc = pc
    d = 1
    while d < NUM_EXPERTS:
        inc = inc + jnp.concatenate(
            [jnp.zeros((1, d), jnp.float32), inc[:, :-d]], axis=1)
        d *= 2
    off = inc - pc
    total = jnp.sum(pc)

    pos0 = jnp.sum(f1 * (off + rank0), axis=-1, keepdims=True)
    pos1 = jnp.sum(f2 * (off + rank1), axis=-1, keepdims=True)
    pos0_ref[...] = pos0.astype(jnp.int32)
    pos1_ref[...] = pos1.astype(jnp.int32)
    m0_ref[...] = mult1
    m1_ref[...] = mult2

    bid = lax.broadcasted_iota(jnp.int32, (BE_PAD, NUM_EXPERTS), 0)
    rowstart = (bid * BLK).astype(jnp.float32)
    cntge = jnp.sum((rowstart >= off).astype(jnp.int32), axis=-1,
                    keepdims=True)
    active = rowstart[:, :1] < total
    be_ref[...] = jnp.where(active, cntge - 1, -1)


def _routing(scores):
    return pl.pallas_call(
        _routing_kernel,
        out_shape=(
            jax.ShapeDtypeStruct((TOKENS, 1), jnp.int32),
            jax.ShapeDtypeStruct((TOKENS, 1), jnp.int32),
            jax.ShapeDtypeStruct((TOKENS, 1), jnp.float32),
            jax.ShapeDtypeStruct((TOKENS, 1), jnp.float32),
            jax.ShapeDtypeStruct((BE_PAD, 1), jnp.int32),
        ),
    )(scores)


def _sc_dispatch(x, pos0, pos1):
    mesh = plsc.VectorSubcoreMesh(core_axis_name="c", subcore_axis_name="s")

    @functools.partial(
        pl.kernel, mesh=mesh,
        out_type=jax.ShapeDtypeStruct((PSLOTS, HIDDEN), jnp.float32),
        scratch_types=[
            pltpu.VMEM((TPW,), jnp.int32),
            pltpu.VMEM((TPW,), jnp.int32),
            pltpu.VMEM((TPW, HIDDEN), jnp.float32),
            pltpu.SemaphoreType.DMA,
        ],
    )
    def k(x_hbm, p0_hbm, p1_hbm, xs_hbm, i0, i1, xb, sem):
        wid = lax.axis_index("s") * 2 + lax.axis_index("c")
        base = wid * TPW
        pltpu.sync_copy(x_hbm.at[pl.ds(base, TPW)], xb)
        pltpu.sync_copy(p0_hbm.at[pl.ds(base, TPW)], i0)
        pltpu.sync_copy(p1_hbm.at[pl.ds(base, TPW)], i1)
        pltpu.async_copy(xb, xs_hbm.at[i0], sem).wait()
        pltpu.async_copy(xb, xs_hbm.at[i1], sem).wait()

    return k(x, pos0, pos1)


def _grouped_kernel(be_ref, xs_ref, w1_ref, w3_ref, w2_ref, ys_ref):
    n = pl.program_id(0)
    b = pl.program_id(1)
    e = be_ref[b]

    @pl.when(e >= 0)
    def _():
        row = pl.multiple_of(b * BLK, BLK)
        xb = xs_ref[...].astype(jnp.bfloat16)
        gate = lax.dot_general(xb, w1_ref[0].astype(jnp.bfloat16),
                               (((1,), (1,)), ((), ())),
                               preferred_element_type=jnp.float32)
        up = lax.dot_general(xb, w3_ref[0].astype(jnp.bfloat16),
                             (((1,), (1,)), ((), ())),
                             preferred_element_type=jnp.float32)
        h = (gate * jax.nn.sigmoid(gate) * up).astype(jnp.bfloat16)
        part = lax.dot_general(h, w2_ref[0].astype(jnp.bfloat16),
                               (((1,), (1,)), ((), ())),
                               preferred_element_type=jnp.float32)

        @pl.when(n == 0)
        def _w():
            ys_ref[pl.ds(row, BLK), :] = part

        @pl.when(n > 0)
        def _a():
            ys_ref[pl.ds(row, BLK), :] += part


def _grouped(be, xs, w1, w3, w2):
    grid_spec = pltpu.PrefetchScalarGridSpec(
        num_scalar_prefetch=1,
        grid=(NI, NBLK),
        in_specs=[
            pl.BlockSpec((BLK, HIDDEN), lambda n, b, be: (b, 0)),
            pl.BlockSpec((1, N_CHUNK, HIDDEN),
                         lambda n, b, be: (jnp.maximum(be[b], 0), n, 0)),
            pl.BlockSpec((1, N_CHUNK, HIDDEN),
                         lambda n, b, be: (jnp.maximum(be[b], 0), n, 0)),
            pl.BlockSpec((1, HIDDEN, N_CHUNK),
                         lambda n, b, be: (jnp.maximum(be[b], 0), 0, n)),
        ],
        out_specs=pl.BlockSpec((PSLOTS, HIDDEN), lambda n, b, be: (0, 0)),
    )
    return pl.pallas_call(
        _grouped_kernel,
        grid_spec=grid_spec,
        out_shape=jax.ShapeDtypeStruct((PSLOTS, HIDDEN), jnp.float32),
        compiler_params=pltpu.CompilerParams(
            dimension_semantics=("arbitrary", "arbitrary"),
        ),
    )(be, xs, w1, w3, w2)


def _sc_combine(ys, pos0, pos1):
    mesh = plsc.VectorSubcoreMesh(core_axis_name="c", subcore_axis_name="s")

    @functools.partial(
        pl.kernel, mesh=mesh,
        out_type=(
            jax.ShapeDtypeStruct((TOKENS, HIDDEN), jnp.float32),
            jax.ShapeDtypeStruct((TOKENS, HIDDEN), jnp.float32),
        ),
        scratch_types=[
            pltpu.VMEM((TPW,), jnp.int32),
            pltpu.VMEM((TPW,), jnp.int32),
            pltpu.VMEM((TPW, HIDDEN), jnp.float32),
            pltpu.SemaphoreType.DMA,
        ],
    )
    def k(ys_hbm, p0_hbm, p1_hbm, g0_hbm, g1_hbm, i0, i1, buf, sem):
        wid = lax.axis_index("s") * 2 + lax.axis_index("c")
        base = wid * TPW
        pltpu.sync_copy(p0_hbm.at[pl.ds(base, TPW)], i0)
        pltpu.async_copy(ys_hbm.at[i0], buf, sem).wait()
        pltpu.sync_copy(buf, g0_hbm.at[pl.ds(base, TPW)])
        pltpu.sync_copy(p1_hbm.at[pl.ds(base, TPW)], i1)
        pltpu.async_copy(ys_hbm.at[i1], buf, sem).wait()
        pltpu.sync_copy(buf, g1_hbm.at[pl.ds(base, TPW)])

    return k(ys, pos0, pos1)


def _combine_kernel(g0_ref, g1_ref, m0_ref, m1_ref, out_ref):
    out_ref[...] = m0_ref[...] * g0_ref[...] + m1_ref[...] * g1_ref[...]


def _combine(g0, g1, m0, m1):
    tb = 256
    return pl.pallas_call(
        _combine_kernel,
        grid=(TOKENS // tb,),
        in_specs=[
            pl.BlockSpec((tb, HIDDEN), lambda i: (i, 0)),
            pl.BlockSpec((tb, HIDDEN), lambda i: (i, 0)),
            pl.BlockSpec((tb, 1), lambda i: (i, 0)),
            pl.BlockSpec((tb, 1), lambda i: (i, 0)),
        ],
        out_specs=pl.BlockSpec((tb, HIDDEN), lambda i: (i, 0)),
        out_shape=jax.ShapeDtypeStruct((TOKENS, HIDDEN), jnp.float32),
    )(g0, g1, m0, m1)


def kernel(hidden_states, w_gate, w1, w2, w3):
    x = hidden_states.reshape(-1, HIDDEN)
    scores = x @ w_gate.T
    pos0c, pos1c, m0, m1, bec = _routing(scores)
    pos0 = pos0c.reshape(TOKENS)
    pos1 = pos1c.reshape(TOKENS)
    be = bec.reshape(BE_PAD)[:NBLK]
    xs = _sc_dispatch(x, pos0, pos1)
    ys = _grouped(be, xs, w1, w3, w2)
    g0, g1 = _sc_combine(ys, pos0, pos1)
    out = _combine(g0, g1, m0, m1)
    return out.reshape(hidden_states.shape)
